# baseline (speedup 1.0000x reference)
import functools

import jax
import jax.numpy as jnp
import numpy as np
from jax import lax
from jax.experimental import pallas as pl
from jax.experimental.pallas import tpu as pltpu
from jax.experimental.pallas import tpu_sc as plsc

_L = 16
_NC = 2
_NS = 16
_NW = _NC * _NS
_T = 24
_IDX_CH = 128

_PERM_CACHE = {}


def _perm_np(n: int):
    if n not in _PERM_CACHE:
        try:
            cpu = jax.devices("cpu")[0]
            with jax.default_device(cpu):
                p = jax.random.permutation(jax.random.key(42), n)
                _PERM_CACHE[n] = np.asarray(p, dtype=np.int32)
        except Exception:
            _PERM_CACHE[n] = None
    return _PERM_CACHE[n]


def _chunks(total: int, ch: int):
    out = []
    off = 0
    while off < total:
        sz = min(ch, total - off)
        out.append((off, sz))
        off += sz
    return out


@functools.lru_cache(maxsize=None)
def _build_sc_call(B: int, N: int, D: int, S: int):
    assert D % _L == 0
    nvr = D // _L
    P = -(-B // (_NW * _T)) * _T
    BP = P * _NW
    NCH = P // _T
    BT = B % _T
    assert NCH >= 3
    scale = 1.0 / S

    mesh = plsc.VectorSubcoreMesh(
        core_axis_name="c", subcore_axis_name="s",
        num_cores=_NC, num_subcores=_NS)

    out_t = jax.ShapeDtypeStruct((B, D), jnp.float32)

    @functools.partial(
        pl.kernel,
        out_type=(out_t,) * 4,
        mesh=mesh,
        scratch_types=[
            pltpu.VMEM((S * P,), jnp.int32),
            pltpu.VMEM((S * P,), jnp.int32),
            pltpu.VMEM((P,), jnp.int32),
            pltpu.VMEM((P,), jnp.int32),
            pltpu.VMEM((2, S, _T, D), jnp.float32),
            pltpu.VMEM((2, _T, D), jnp.float32),
            pltpu.SemaphoreType.DMA,
            pltpu.SemaphoreType.DMA,
            pltpu.SemaphoreType.DMA,
            pltpu.SemaphoreType.DMA,
            pltpu.SemaphoreType.DMA,
        ],
    )
    def sc_body(nodes_hbm, neigh_hbm, feat_hbm, perm_hbm,
                to_hbm, shto_hbm, sk_hbm, shsk_hbm,
                ng_idx, ng_shuf, nd_idx, nd_shuf, gbuf, obuf,
                gsem0, gsem1, gsem2, osem, csem):
        gsems = (gsem0, gsem1, gsem2)
        wid = lax.axis_index("s") * _NC + lax.axis_index("c")
        base = wid * P

        pltpu.sync_copy(nodes_hbm.at[pl.ds(base, P)], nd_idx)
        for j in range(S):
            pltpu.sync_copy(neigh_hbm.at[pl.ds(j * BP + base, P)],
                            ng_idx.at[pl.ds(j * P, P)])

        nd_cps = []
        for off, sz in _chunks(P, _IDX_CH):
            cp = pltpu.make_async_copy(
                perm_hbm.at[nd_idx.at[pl.ds(off, sz)]],
                nd_shuf.at[pl.ds(off, sz)], csem)
            cp.start()
            nd_cps.append(cp)

        def write_out(out_hbm, c, stage_ref):
            @pl.when(base + c * _T + _T <= B)
            def _():
                pltpu.make_async_copy(
                    stage_ref, out_hbm.at[pl.ds(base + c * _T, _T)],
                    osem).start()
            if BT:
                @pl.when(jnp.logical_and(base + c * _T < B,
                                         base + c * _T + _T > B))
                def _():
                    pltpu.sync_copy(
                        stage_ref.at[pl.ds(0, BT)],
                        out_hbm.at[pl.ds(base + c * _T, BT)])

        def drain_out(out_hbm, c, stage_ref):
            @pl.when(base + c * _T + _T <= B)
            def _():
                pltpu.make_async_copy(
                    stage_ref, out_hbm.at[pl.ds(base + c * _T, _T)],
                    osem).wait()

        def skip_path(idx_ref, out_hbm):
            def fire(c, s):
                off = pl.multiple_of(c * _T, 8)
                pltpu.make_async_copy(
                    feat_hbm.at[idx_ref.at[pl.ds(off, _T)]],
                    gbuf.at[0, s], gsems[s]).start()

            def drain(c, s):
                off = pl.multiple_of(c * _T, 8)
                pltpu.make_async_copy(
                    feat_hbm.at[idx_ref.at[pl.ds(off, _T)]],
                    gbuf.at[0, s], gsems[s]).wait()

            fire(0, 0)
            fire(1, 1)
            TRI = NCH // 3

            @pl.loop(0, TRI)
            def _tri(t):
                c0 = pl.multiple_of(t * 3, 3)
                for i in range(3):
                    c = c0 + i
                    s = i
                    cf = c + 2
                    if True:
                        @pl.when(c >= 1)
                        def _(c=c):
                            drain_out(out_hbm, c - 1, gbuf.at[0, (i - 1) % 3])

                        @pl.when(cf < NCH)
                        def _(cf=cf):
                            fire(cf, (i + 2) % 3)
                        drain(c, s)
                        write_out(out_hbm, c, gbuf.at[0, s])

            for c in range(TRI * 3, NCH):
                s = c % 3
                if c >= 1:
                    drain_out(out_hbm, c - 1, gbuf.at[0, (c - 1) % 3])
                drain(c, s)
                write_out(out_hbm, c, gbuf.at[0, s])
            drain_out(out_hbm, NCH - 1, gbuf.at[0, (NCH - 1) % 3])

        skip_path(nd_idx, sk_hbm)
        for cp in nd_cps:
            cp.wait()
        skip_path(nd_shuf, shsk_hbm)

        ng_total = S * P
        GRP = 8
        full = (ng_total // _IDX_CH // GRP) * GRP

        @pl.loop(0, full // GRP)
        def _compose(g):
            goff = pl.multiple_of(g * (GRP * _IDX_CH), GRP * _IDX_CH)
            cps = []
            for i in range(GRP):
                off = goff + i * _IDX_CH
                cp = pltpu.make_async_copy(
                    perm_hbm.at[ng_idx.at[pl.ds(off, _IDX_CH)]],
                    ng_shuf.at[pl.ds(off, _IDX_CH)], csem)
                cp.start()
                cps.append(cp)
            for cp in cps:
                cp.wait()

        cps = []
        for off, sz in _chunks(ng_total - full * _IDX_CH, _IDX_CH):
            cp = pltpu.make_async_copy(
                perm_hbm.at[ng_idx.at[pl.ds(full * _IDX_CH + off, sz)]],
                ng_shuf.at[pl.ds(full * _IDX_CH + off, sz)], csem)
            cp.start()
            cps.append(cp)
        for cp in cps:
            cp.wait()

        def neigh_path(idx_ref, out_hbm):
            def fire(c, b):
                off = pl.multiple_of(c * _T, 8)
                for j in range(S):
                    pltpu.make_async_copy(
                        feat_hbm.at[idx_ref.at[pl.ds(j * P + off, _T)]],
                        gbuf.at[b, j], gsems[b]).start()

            def drain(c, b):
                off = pl.multiple_of(c * _T, 8)
                for j in range(S):
                    pltpu.make_async_copy(
                        feat_hbm.at[idx_ref.at[pl.ds(j * P + off, _T)]],
                        gbuf.at[b, j], gsems[b]).wait()

            def compute(b):
                @pl.loop(0, _T // 4)
                def _rows(rb):
                    r0 = pl.multiple_of(rb * 4, 4)
                    for r in range(4):
                        row = r0 + r
                        acc = [gbuf[b, 0, row, pl.ds(cc * _L, _L)]
                               for cc in range(nvr)]
                        for j in range(1, S):
                            for cc in range(nvr):
                                acc[cc] = acc[cc] + gbuf[b, j, row,
                                                         pl.ds(cc * _L, _L)]
                        for cc in range(nvr):
                            obuf[b, row, pl.ds(cc * _L, _L)] = (
                                acc[cc] * jnp.float32(scale))

            def phase(c, b, static):
                drain(c, b)
                if static:
                    if c >= 2:
                        drain_out(out_hbm, c - 2, obuf.at[b])
                else:
                    @pl.when(c >= 2)
                    def _():
                        drain_out(out_hbm, c - 2, obuf.at[b])
                compute(b)
                write_out(out_hbm, c, obuf.at[b])
                if static:
                    if c + 2 < NCH:
                        fire(c + 2, b)
                else:
                    @pl.when(c + 2 < NCH)
                    def _():
                        fire(c + 2, b)

            fire(0, 0)
            fire(1, 1)
            PAIRS = NCH // 2

            @pl.loop(0, PAIRS)
            def _pair(h):
                c0 = pl.multiple_of(h * 2, 2)
                phase(c0, 0, False)
                phase(c0 + 1, 1, False)

            for c in range(PAIRS * 2, NCH):
                phase(c, c % 2, True)
            drain_out(out_hbm, NCH - 2, obuf.at[(NCH - 2) % 2])
            drain_out(out_hbm, NCH - 1, obuf.at[(NCH - 1) % 2])

        neigh_path(ng_idx, to_hbm)
        neigh_path(ng_shuf, shto_hbm)

    return sc_body, P, BP


def kernel(nodes, neigh_idx, features):
    B = nodes.shape[0]
    N, D = features.shape
    S = neigh_idx.shape[1]
    sc_call, P, BP = _build_sc_call(B, N, D, S)
    perm_host = _perm_np(N)
    if perm_host is not None:
        perm = jnp.asarray(perm_host)
    else:
        perm = jax.random.permutation(jax.random.key(42), N).astype(jnp.int32)
    pad = BP - B
    nodes_p = jnp.concatenate([nodes, jnp.zeros((pad,), jnp.int32)])
    neigh_t = jnp.concatenate(
        [neigh_idx, jnp.zeros((pad, S), jnp.int32)]).T.reshape(-1)
    to_f, shto_f, sk_f, shsk_f = sc_call(nodes_p, neigh_t, features, perm)
    return (to_f, shto_f, sk_f, shsk_f)


# speedup vs baseline: 1.0275x; 1.0275x over previous
import functools

import jax
import jax.numpy as jnp
import numpy as np
from jax import lax
from jax.experimental import pallas as pl
from jax.experimental.pallas import tpu as pltpu
from jax.experimental.pallas import tpu_sc as plsc

_L = 16
_NC = 2
_NS = 16
_NW = _NC * _NS
_CR = 12
_GROUP = 4 * _CR
_OCT = 8 * _CR
_SK = 64
_IDX_CH = 128

_PERM_CACHE = {}


def _perm_np(n: int):
    if n not in _PERM_CACHE:
        try:
            cpu = jax.devices("cpu")[0]
            with jax.default_device(cpu):
                p = jax.random.permutation(jax.random.key(42), n)
                _PERM_CACHE[n] = np.asarray(p, dtype=np.int32)
        except Exception:
            _PERM_CACHE[n] = None
    return _PERM_CACHE[n]


def _chunks(total: int, ch: int):
    out = []
    off = 0
    while off < total:
        sz = min(ch, total - off)
        out.append((off, sz))
        off += sz
    return out


@functools.lru_cache(maxsize=None)
def _build_sc_call(B: int, N: int, D: int, S: int):
    assert D % _L == 0 and S * _CR <= _IDX_CH
    nvr = D // _L
    P = -(-B // (_NW * _GROUP)) * _GROUP
    BP = P * _NW
    NCH = P // _CR
    NGRP = P // _GROUP
    OCTS = NCH // 8
    assert NCH % 4 == 0
    BT = B % _GROUP
    assert BT % 8 == 0 and B % 8 == 0
    sk_chunks = _chunks(P, _SK)
    assert (B - (_NW - 1) * P) % _SK == 0 or (_NW - 1) * P >= B
    scale = 1.0 / S

    mesh = plsc.VectorSubcoreMesh(
        core_axis_name="c", subcore_axis_name="s",
        num_cores=_NC, num_subcores=_NS)

    out_t = jax.ShapeDtypeStruct((B, D), jnp.float32)

    @functools.partial(
        pl.kernel,
        out_type=(out_t,) * 4,
        mesh=mesh,
        scratch_types=[
            pltpu.VMEM((S * P,), jnp.int32),
            pltpu.VMEM((S * P,), jnp.int32),
            pltpu.VMEM((P,), jnp.int32),
            pltpu.VMEM((P,), jnp.int32),
            pltpu.VMEM((2 * S * _CR, D), jnp.float32),
            pltpu.VMEM((2, _GROUP, D), jnp.float32),
            pltpu.SemaphoreType.DMA,
            pltpu.SemaphoreType.DMA,
            pltpu.SemaphoreType.DMA,
            pltpu.SemaphoreType.DMA,
            pltpu.SemaphoreType.DMA,
        ],
    )
    def sc_body(nodes_hbm, neigh_hbm, feat_hbm, perm_hbm,
                to_hbm, shto_hbm, sk_hbm, shsk_hbm,
                ng_idx, ng_shuf, nd_idx, nd_shuf, gbuf, obuf,
                gsem0, gsem1, gsem2, osem, csem):
        gsems = (gsem0, gsem1, gsem2)
        wid = lax.axis_index("s") * _NC + lax.axis_index("c")
        base = wid * P

        pltpu.sync_copy(nodes_hbm.at[pl.ds(base, P)], nd_idx)
        pltpu.sync_copy(neigh_hbm.at[pl.ds(base * S, S * P)], ng_idx)

        nd_cps = []
        for off, sz in _chunks(P, _IDX_CH):
            cp = pltpu.make_async_copy(
                perm_hbm.at[nd_idx.at[pl.ds(off, sz)]],
                nd_shuf.at[pl.ds(off, sz)], csem)
            cp.start()
            nd_cps.append(cp)

        def skip_path(idx_ref, out_hbm):
            nsk = len(sk_chunks)

            def slot_ref(s, sz):
                return gbuf.at[pl.ds(s * _SK, sz)]

            def fire_s(ci, s):
                off, sz = sk_chunks[ci]
                pltpu.make_async_copy(
                    feat_hbm.at[idx_ref.at[pl.ds(off, sz)]],
                    slot_ref(s, sz), gsems[s]).start()

            def drain_s(ci, s):
                off, sz = sk_chunks[ci]
                pltpu.make_async_copy(
                    feat_hbm.at[idx_ref.at[pl.ds(off, sz)]],
                    slot_ref(s, sz), gsems[s]).wait()

            def write_s(ci, s):
                off, sz = sk_chunks[ci]

                @pl.when(base + off + sz <= B)
                def _():
                    pltpu.make_async_copy(
                        slot_ref(s, sz),
                        out_hbm.at[pl.ds(base + off, sz)], osem).start()

            def drain_w(ci, s):
                off, sz = sk_chunks[ci]

                @pl.when(base + off + sz <= B)
                def _():
                    pltpu.make_async_copy(
                        slot_ref(s, sz),
                        out_hbm.at[pl.ds(base + off, sz)], osem).wait()

            fire_s(0, 0)
            if nsk > 1:
                fire_s(1, 1)
            for ci in range(nsk):
                s = ci % 3
                if ci >= 1:
                    drain_w(ci - 1, (ci - 1) % 3)
                if ci + 2 < nsk:
                    fire_s(ci + 2, (ci + 2) % 3)
                drain_s(ci, s)
                write_s(ci, s)
            drain_w(nsk - 1, (nsk - 1) % 3)

        skip_path(nd_idx, sk_hbm)
        for cp in nd_cps:
            cp.wait()
        skip_path(nd_shuf, shsk_hbm)

        ng_total = S * P
        GRP = 8
        full = (ng_total // _IDX_CH // GRP) * GRP

        @pl.loop(0, full // GRP)
        def _compose(g):
            goff = pl.multiple_of(g * (GRP * _IDX_CH), GRP * _IDX_CH)
            cps = []
            for i in range(GRP):
                off = goff + i * _IDX_CH
                cp = pltpu.make_async_copy(
                    perm_hbm.at[ng_idx.at[pl.ds(off, _IDX_CH)]],
                    ng_shuf.at[pl.ds(off, _IDX_CH)], csem)
                cp.start()
                cps.append(cp)
            for cp in cps:
                cp.wait()

        cps = []
        for off, sz in _chunks(ng_total - full * _IDX_CH, _IDX_CH):
            cp = pltpu.make_async_copy(
                perm_hbm.at[ng_idx.at[pl.ds(full * _IDX_CH + off, sz)]],
                ng_shuf.at[pl.ds(full * _IDX_CH + off, sz)], csem)
            cp.start()
            cps.append(cp)
        for cp in cps:
            cp.wait()

        def neigh_path(idx_ref, out_hbm):
            def fire(c, b):
                off = pl.multiple_of(c * (S * _CR), 8)
                pltpu.make_async_copy(
                    feat_hbm.at[idx_ref.at[pl.ds(off, S * _CR)]],
                    gbuf.at[pl.ds(b * S * _CR, S * _CR)], gsems[b]).start()

            def drain(c, b):
                off = pl.multiple_of(c * (S * _CR), 8)
                pltpu.make_async_copy(
                    feat_hbm.at[idx_ref.at[pl.ds(off, S * _CR)]],
                    gbuf.at[pl.ds(b * S * _CR, S * _CR)], gsems[b]).wait()

            def compute(b, gslot, part):
                @pl.loop(0, _CR // 2)
                def _rows(rb):
                    r0 = pl.multiple_of(rb * 2, 2)
                    for r in range(2):
                        row = r0 + r
                        g0 = b * S * _CR + row * S
                        acc = [gbuf[g0, pl.ds(cc * _L, _L)]
                               for cc in range(nvr)]
                        for j in range(1, S):
                            for cc in range(nvr):
                                acc[cc] = acc[cc] + gbuf[g0 + j,
                                                         pl.ds(cc * _L, _L)]
                        for cc in range(nvr):
                            obuf[gslot, part * _CR + row,
                                 pl.ds(cc * _L, _L)] = (
                                acc[cc] * jnp.float32(scale))

            def write_group(g, gslot):
                @pl.when(base + g * _GROUP + _GROUP <= B)
                def _():
                    pltpu.make_async_copy(
                        obuf.at[gslot],
                        out_hbm.at[pl.ds(base + g * _GROUP, _GROUP)],
                        osem).start()
                if BT:
                    @pl.when(jnp.logical_and(base + g * _GROUP < B,
                                             base + g * _GROUP + _GROUP > B))
                    def _():
                        pltpu.sync_copy(
                            obuf.at[gslot, pl.ds(0, BT)],
                            out_hbm.at[pl.ds(base + g * _GROUP, BT)])

            def drain_group(g, gslot):
                @pl.when(base + g * _GROUP + _GROUP <= B)
                def _():
                    pltpu.make_async_copy(
                        obuf.at[gslot],
                        out_hbm.at[pl.ds(base + g * _GROUP, _GROUP)],
                        osem).wait()

            fire(0, 0)
            fire(1, 1)

            @pl.loop(0, OCTS)
            def _oct(u):
                c0 = pl.multiple_of(u * 8, 8)
                g0 = pl.multiple_of(u * 2, 2)
                for i in range(8):
                    c = c0 + i
                    b = i % 2
                    gslot = i // 4
                    if i == 0:
                        @pl.when(g0 >= 2)
                        def _():
                            drain_group(g0 - 2, 0)
                    if i == 4:
                        @pl.when(g0 + 1 >= 2)
                        def _():
                            drain_group(g0 - 1, 1)
                    drain(c, b)
                    compute(b, gslot, i % 4)

                    @pl.when(c + 2 < NCH)
                    def _(c=c, b=b):
                        fire(c + 2, b)
                    if i == 3:
                        write_group(g0, 0)
                    if i == 7:
                        write_group(g0 + 1, 1)

            for c in range(OCTS * 8, NCH):
                i = c - OCTS * 8
                b = c % 2
                g = c // 4
                gslot = g % 2
                if i % 4 == 0:
                    if g >= 2:
                        drain_group(g - 2, gslot)
                drain(c, b)
                compute(b, gslot, c % 4)
                if c + 2 < NCH:
                    fire(c + 2, b)
                if c % 4 == 3:
                    write_group(g, gslot)
            drain_group(NGRP - 2, (NGRP - 2) % 2)
            drain_group(NGRP - 1, (NGRP - 1) % 2)

        neigh_path(ng_idx, to_hbm)
        neigh_path(ng_shuf, shto_hbm)

    return sc_body, P, BP


def kernel(nodes, neigh_idx, features):
    B = nodes.shape[0]
    N, D = features.shape
    S = neigh_idx.shape[1]
    sc_call, P, BP = _build_sc_call(B, N, D, S)
    perm_host = _perm_np(N)
    if perm_host is not None:
        perm = jnp.asarray(perm_host)
    else:
        perm = jax.random.permutation(jax.random.key(42), N).astype(jnp.int32)
    pad = BP - B
    nodes_p = jnp.concatenate([nodes, jnp.zeros((pad,), jnp.int32)])
    neigh_f = jnp.concatenate(
        [neigh_idx, jnp.zeros((pad, S), jnp.int32)]).reshape(-1)
    to_f, shto_f, sk_f, shsk_f = sc_call(nodes_p, neigh_f, features, perm)
    return (to_f, shto_f, sk_f, shsk_f)
